# Initial kernel scaffold; baseline (speedup 1.0000x reference)
import jax
import jax.numpy as jnp
from jax.experimental import pallas as pl


def kernel(x, edge_index, edge_attr, smiles, batch, lin1_w, lin1_b, fc1_w, fc1_b, fc2_w, fc2_b):
    raise NotImplementedError("write your pallas kernel here")


# speedup vs baseline: 5.8452x; 5.8452x over previous
import functools

import jax
import jax.numpy as jnp
from jax import lax
from jax.experimental import pallas as pl
from jax.experimental.pallas import tpu as pltpu
from jax.experimental.pallas import tpu_sc as plsc

N = 10000
E = 320000
D = 128
DE = 16
G = 512
INNER = 512
HID = 50

NC = 2
NS = 16
NW = NC * NS
EPW = E // NW
CH = 80
NCHUNK = EPW // CH
RPT = G // NS


def _sc_body(x_hbm, src_hbm, dst_hbm, eaf_hbm, batch_hbm,
             px_hbm, pa_hbm,
             src_v, dst_v, ge_v, xrows_v, ealin_v, ea_v,
             zx_v, acc_x, acc_a, sem):
    c = lax.axis_index("c")
    s = lax.axis_index("s")
    wid = s * NC + c

    zero16 = jnp.zeros((16,), jnp.float32)
    onehot16 = jnp.where(lax.iota(jnp.int32, 16) == 0, 1.0, 0.0)

    for j in range(RPT):
        for k in range(D // 16):
            zx_v[j, pl.ds(k * 16, 16)] = zero16
    for j in range(CH):
        ea_v[j, pl.ds(16, 16)] = onehot16
        for k in range(2, D // 16):
            ea_v[j, pl.ds(k * 16, 16)] = zero16

    rowbase = s * RPT
    pltpu.sync_copy(zx_v, acc_x.at[pl.ds(rowbase, RPT)])
    pltpu.sync_copy(zx_v, acc_a.at[pl.ds(rowbase, RPT)])
    plsc.subcore_barrier()

    def chunk(i, _):
        base = wid * EPW + i * CH
        pltpu.sync_copy(src_hbm.at[pl.ds(base, CH)], src_v)
        pltpu.sync_copy(dst_hbm.at[pl.ds(base, CH)], dst_v)
        pltpu.sync_copy(eaf_hbm.at[pl.ds(base * DE, CH * DE)], ealin_v)
        pltpu.async_copy(batch_hbm.at[dst_v], ge_v, sem).wait()
        pltpu.async_copy(x_hbm.at[src_v], xrows_v, sem).wait()
        for e in range(CH):
            ea_v[e, pl.ds(0, 16)] = ealin_v[pl.ds(e * DE, 16)]
        pltpu.sync_copy(xrows_v, acc_x.at[ge_v], add=True)
        pltpu.sync_copy(ea_v, acc_a.at[ge_v], add=True)
        return _

    lax.fori_loop(0, NCHUNK, chunk, 0)

    plsc.subcore_barrier()
    outbase = c * G + rowbase
    pltpu.sync_copy(acc_x.at[pl.ds(rowbase, RPT)], px_hbm.at[pl.ds(outbase, RPT)])
    pltpu.sync_copy(acc_a.at[pl.ds(rowbase, RPT)], pa_hbm.at[pl.ds(outbase, RPT)])


@functools.cache
def _sc_aggregate():
    return functools.partial(
        pl.kernel,
        out_type=[
            jax.ShapeDtypeStruct((NC * G, D), jnp.float32),
            jax.ShapeDtypeStruct((NC * G, D), jnp.float32),
        ],
        mesh=plsc.VectorSubcoreMesh(core_axis_name="c", subcore_axis_name="s"),
        compiler_params=pltpu.CompilerParams(needs_layout_passes=False),
        scratch_types=[
            pltpu.VMEM((CH,), jnp.int32),
            pltpu.VMEM((CH,), jnp.int32),
            pltpu.VMEM((CH,), jnp.int32),
            pltpu.VMEM((CH, D), jnp.float32),
            pltpu.VMEM((CH * DE,), jnp.float32),
            pltpu.VMEM((CH, D), jnp.float32),
            pltpu.VMEM((RPT, D), jnp.float32),
            pltpu.VMEM_SHARED((G, D), jnp.float32),
            pltpu.VMEM_SHARED((G, D), jnp.float32),
            pltpu.SemaphoreType.DMA,
        ],
    )(_sc_body)


def _tc_body(px, pa, w1xt, wa_ext, f1t, b1f, f2t, b2r, out):
    dot = functools.partial(jnp.dot, preferred_element_type=jnp.float32,
                            precision=lax.Precision.HIGHEST)
    sx = px[0:G, :] + px[G:2 * G, :]
    sa = pa[0:G, :] + pa[G:2 * G, :]
    mol = dot(sx, w1xt[...]) + dot(sa, wa_ext[...])
    hid = dot(mol, f1t[...]) + b1f[...]
    out[...] = dot(hid, f2t[...]) + b2r[...]


def kernel(x, edge_index, edge_attr, smiles, batch, lin1_w, lin1_b,
           fc1_w, fc1_b, fc2_w, fc2_b):
    src = edge_index[0]
    dst = edge_index[1]
    px, pa = _sc_aggregate()(x, src, dst, edge_attr.reshape(-1), batch)

    hpad = 64
    w1xt = lin1_w[:, :D].T
    wa_ext = (jnp.zeros((D, INNER), jnp.float32)
              .at[:DE].set(lin1_w[:, D:].T)
              .at[DE].set(lin1_b))
    f1t = jnp.zeros((INNER, hpad), jnp.float32).at[:, :HID].set(fc1_w.T)
    b1f = jnp.zeros((1, hpad), jnp.float32).at[0, :HID].set(fc1_b)
    f2t = jnp.zeros((hpad, 128), jnp.float32).at[:HID, 0].set(fc2_w[0])
    b2r = jnp.zeros((1, 128), jnp.float32).at[0, 0].set(fc2_b[0])

    outf = pl.pallas_call(
        _tc_body,
        out_shape=jax.ShapeDtypeStruct((G, 128), jnp.float32),
    )(px, pa, w1xt, wa_ext, f1t, b1f, f2t, b2r)
    return outf[:, :1]


# speedup vs baseline: 8.5742x; 1.4669x over previous
import functools

import jax
import jax.numpy as jnp
from jax import lax
from jax.experimental import pallas as pl
from jax.experimental.pallas import tpu as pltpu
from jax.experimental.pallas import tpu_sc as plsc

N = 10000
E = 320000
D = 128
DE = 16
G = 512
INNER = 512
HID = 50

NC = 2
NS = 16
NW = NC * NS
EPW = E // NW
CH = 80
NCHUNK = EPW // CH
RPT = G // NS


def _sc_body(x_hbm, src_hbm, dst_hbm, eaf_hbm, batch_hbm,
             px_hbm, pa_hbm,
             src_a, dst_a, ge_a, xrows_a, ealin_a,
             src_b, dst_b, ge_b, xrows_b, ealin_b,
             ea_v, zx_v, acc_x, acc_a, sem_a, sem_b):
    c = lax.axis_index("c")
    s = lax.axis_index("s")
    wid = s * NC + c

    zero16 = jnp.zeros((16,), jnp.float32)
    onehot16 = jnp.where(lax.iota(jnp.int32, 16) == 0, 1.0, 0.0)

    for j in range(RPT):
        for k in range(D // 16):
            zx_v[j, pl.ds(k * 16, 16)] = zero16
    for j in range(CH):
        ea_v[j, pl.ds(16, 16)] = onehot16
        for k in range(2, D // 16):
            ea_v[j, pl.ds(k * 16, 16)] = zero16

    rowbase = s * RPT
    pltpu.sync_copy(zx_v, acc_x.at[pl.ds(rowbase, RPT)])
    pltpu.sync_copy(zx_v, acc_a.at[pl.ds(rowbase, RPT)])
    plsc.subcore_barrier()

    bufs = ((src_a, dst_a, ge_a, xrows_a, ealin_a, sem_a),
            (src_b, dst_b, ge_b, xrows_b, ealin_b, sem_b))

    def load(j, buf):
        src_v, dst_v, ge_v, xrows_v, ealin_v, sem = buf
        base = wid * EPW + j * CH
        pltpu.sync_copy(src_hbm.at[pl.ds(base, CH)], src_v)
        pltpu.sync_copy(dst_hbm.at[pl.ds(base, CH)], dst_v)
        pltpu.sync_copy(eaf_hbm.at[pl.ds(base * DE, CH * DE)], ealin_v)
        pltpu.async_copy(batch_hbm.at[dst_v], ge_v, sem)
        pltpu.async_copy(x_hbm.at[src_v], xrows_v, sem)

    def process(buf):
        src_v, dst_v, ge_v, xrows_v, ealin_v, sem = buf
        pltpu.make_async_copy(batch_hbm.at[dst_v], ge_v, sem).wait()
        pltpu.make_async_copy(x_hbm.at[src_v], xrows_v, sem).wait()
        for e in range(CH):
            ea_v[e, pl.ds(0, 16)] = ealin_v[pl.ds(e * DE, 16)]
        pltpu.sync_copy(xrows_v, acc_x.at[ge_v], add=True)
        pltpu.sync_copy(ea_v, acc_a.at[ge_v], add=True)

    load(0, bufs[0])

    def pair(i2, carry):
        j0 = 2 * i2
        load(j0 + 1, bufs[1])
        process(bufs[0])

        @pl.when(j0 + 2 < NCHUNK)
        def _prefetch():
            load(j0 + 2, bufs[0])

        process(bufs[1])
        return carry

    lax.fori_loop(0, NCHUNK // 2, pair, 0)
    process(bufs[0])

    plsc.subcore_barrier()
    outbase = c * G + rowbase
    pltpu.sync_copy(acc_x.at[pl.ds(rowbase, RPT)], px_hbm.at[pl.ds(outbase, RPT)])
    pltpu.sync_copy(acc_a.at[pl.ds(rowbase, RPT)], pa_hbm.at[pl.ds(outbase, RPT)])


@functools.cache
def _sc_aggregate():
    return functools.partial(
        pl.kernel,
        out_type=[
            jax.ShapeDtypeStruct((NC * G, D), jnp.float32),
            jax.ShapeDtypeStruct((NC * G, D), jnp.float32),
        ],
        mesh=plsc.VectorSubcoreMesh(core_axis_name="c", subcore_axis_name="s"),
        compiler_params=pltpu.CompilerParams(needs_layout_passes=False),
        scratch_types=(
            [pltpu.VMEM((CH,), jnp.int32),
             pltpu.VMEM((CH,), jnp.int32),
             pltpu.VMEM((CH,), jnp.int32),
             pltpu.VMEM((CH, D), jnp.float32),
             pltpu.VMEM((CH * DE,), jnp.float32)]
            * 2
            + [pltpu.VMEM((CH, D), jnp.float32),
               pltpu.VMEM((RPT, D), jnp.float32),
               pltpu.VMEM_SHARED((G, D), jnp.float32),
               pltpu.VMEM_SHARED((G, D), jnp.float32),
               pltpu.SemaphoreType.DMA,
               pltpu.SemaphoreType.DMA]
        ),
    )(_sc_body)


def _tc_body(px, pa, w1xt, wa_ext, f1t, b1f, f2t, b2r, out):
    dot = functools.partial(jnp.dot, preferred_element_type=jnp.float32,
                            precision=lax.Precision.HIGHEST)
    sx = px[0:G, :] + px[G:2 * G, :]
    sa = pa[0:G, :] + pa[G:2 * G, :]
    mol = dot(sx, w1xt[...]) + dot(sa, wa_ext[...])
    hid = dot(mol, f1t[...]) + b1f[...]
    out[...] = dot(hid, f2t[...]) + b2r[...]


def kernel(x, edge_index, edge_attr, smiles, batch, lin1_w, lin1_b,
           fc1_w, fc1_b, fc2_w, fc2_b):
    src = edge_index[0]
    dst = edge_index[1]
    px, pa = _sc_aggregate()(x, src, dst, edge_attr.reshape(-1), batch)

    hpad = 64
    w1xt = lin1_w[:, :D].T
    wa_ext = (jnp.zeros((D, INNER), jnp.float32)
              .at[:DE].set(lin1_w[:, D:].T)
              .at[DE].set(lin1_b))
    f1t = jnp.zeros((INNER, hpad), jnp.float32).at[:, :HID].set(fc1_w.T)
    b1f = jnp.zeros((1, hpad), jnp.float32).at[0, :HID].set(fc1_b)
    f2t = jnp.zeros((hpad, 128), jnp.float32).at[:HID, 0].set(fc2_w[0])
    b2r = jnp.zeros((1, 128), jnp.float32).at[0, 0].set(fc2_b[0])

    outf = pl.pallas_call(
        _tc_body,
        out_shape=jax.ShapeDtypeStruct((G, 128), jnp.float32),
    )(px, pa, w1xt, wa_ext, f1t, b1f, f2t, b2r)
    return outf[:, :1]


# speedup vs baseline: 12.9554x; 1.5110x over previous
import functools

import jax
import jax.numpy as jnp
from jax import lax
from jax.experimental import pallas as pl
from jax.experimental.pallas import tpu as pltpu
from jax.experimental.pallas import tpu_sc as plsc

N = 10000
E = 320000
D = 128
DE = 16
G = 512
INNER = 512
HID = 50

NC = 2
NS = 16
NW = NC * NS
EPW = E // NW
CH = 80
NCHUNK = EPW // CH
RPT = G // NS


def _sc_body(x_hbm, src_hbm, dst_hbm, eaf_hbm, batch_hbm,
             px_hbm, pa_hbm,
             ge_a, xrows_a, ealin_a,
             ge_b, xrows_b, ealin_b,
             src_all, dst_all,
             ea_v, zx_v, acc_x, acc_a, sem_a, sem_b):
    c = lax.axis_index("c")
    s = lax.axis_index("s")
    wid = s * NC + c

    zero16 = jnp.zeros((16,), jnp.float32)
    onehot16 = jnp.where(lax.iota(jnp.int32, 16) == 0, 1.0, 0.0)

    for j in range(RPT):
        for k in range(D // 16):
            zx_v[j, pl.ds(k * 16, 16)] = zero16
    for j in range(CH):
        ea_v[j, pl.ds(16, 16)] = onehot16
        for k in range(2, D // 16):
            ea_v[j, pl.ds(k * 16, 16)] = zero16

    rowbase = s * RPT
    pltpu.sync_copy(zx_v, acc_x.at[pl.ds(rowbase, RPT)])
    pltpu.sync_copy(zx_v, acc_a.at[pl.ds(rowbase, RPT)])
    ebase = wid * EPW
    pltpu.sync_copy(src_hbm.at[pl.ds(ebase, EPW)], src_all)
    pltpu.sync_copy(dst_hbm.at[pl.ds(ebase, EPW)], dst_all)
    plsc.subcore_barrier()

    bufs = ((ge_a, xrows_a, ealin_a, sem_a),
            (ge_b, xrows_b, ealin_b, sem_b))

    def load(j, buf):
        ge_v, xrows_v, ealin_v, sem = buf
        pltpu.async_copy(eaf_hbm.at[pl.ds((ebase + j * CH) * DE, CH * DE)],
                         ealin_v, sem)
        pltpu.async_copy(batch_hbm.at[dst_all.at[pl.ds(j * CH, CH)]], ge_v, sem)
        pltpu.async_copy(x_hbm.at[src_all.at[pl.ds(j * CH, CH)]], xrows_v, sem)

    def process(buf):
        ge_v, xrows_v, ealin_v, sem = buf
        pltpu.make_async_copy(eaf_hbm.at[pl.ds(0, CH * DE)], ealin_v, sem).wait()
        pltpu.make_async_copy(batch_hbm.at[ge_v], ge_v, sem).wait()
        pltpu.make_async_copy(x_hbm.at[ge_v], xrows_v, sem).wait()
        for e in range(CH):
            ea_v[e, pl.ds(0, 16)] = ealin_v[pl.ds(e * DE, 16)]
        pltpu.sync_copy(xrows_v, acc_x.at[ge_v], add=True)
        pltpu.sync_copy(ea_v, acc_a.at[ge_v], add=True)

    load(0, bufs[0])

    def pair(i2, carry):
        j0 = 2 * i2
        load(j0 + 1, bufs[1])
        process(bufs[0])

        @pl.when(j0 + 2 < NCHUNK)
        def _prefetch():
            load(j0 + 2, bufs[0])

        process(bufs[1])
        return carry

    lax.fori_loop(0, NCHUNK // 2, pair, 0)
    process(bufs[0])

    plsc.subcore_barrier()
    outbase = c * G + rowbase
    pltpu.sync_copy(acc_x.at[pl.ds(rowbase, RPT)], px_hbm.at[pl.ds(outbase, RPT)])
    pltpu.sync_copy(acc_a.at[pl.ds(rowbase, RPT)], pa_hbm.at[pl.ds(outbase, RPT)])


@functools.cache
def _sc_aggregate():
    return functools.partial(
        pl.kernel,
        out_type=[
            jax.ShapeDtypeStruct((NC * G, D), jnp.float32),
            jax.ShapeDtypeStruct((NC * G, D), jnp.float32),
        ],
        mesh=plsc.VectorSubcoreMesh(core_axis_name="c", subcore_axis_name="s"),
        compiler_params=pltpu.CompilerParams(needs_layout_passes=False),
        scratch_types=(
            [pltpu.VMEM((CH,), jnp.int32),
             pltpu.VMEM((CH, D), jnp.float32),
             pltpu.VMEM((CH * DE,), jnp.float32)]
            * 2
            + [pltpu.VMEM((EPW,), jnp.int32),
               pltpu.VMEM((EPW,), jnp.int32),
               pltpu.VMEM((CH, D), jnp.float32),
               pltpu.VMEM((RPT, D), jnp.float32),
               pltpu.VMEM_SHARED((G, D), jnp.float32),
               pltpu.VMEM_SHARED((G, D), jnp.float32),
               pltpu.SemaphoreType.DMA,
               pltpu.SemaphoreType.DMA]
        ),
    )(_sc_body)


def _tc_body(px, pa, w1xt, wa_ext, f1t, b1f, f2t, b2r, out):
    dot = functools.partial(jnp.dot, preferred_element_type=jnp.float32,
                            precision=lax.Precision.HIGHEST)
    sx = px[0:G, :] + px[G:2 * G, :]
    sa = pa[0:G, :] + pa[G:2 * G, :]
    mol = dot(sx, w1xt[...]) + dot(sa, wa_ext[...])
    hid = dot(mol, f1t[...]) + b1f[...]
    out[...] = dot(hid, f2t[...]) + b2r[...]


def kernel(x, edge_index, edge_attr, smiles, batch, lin1_w, lin1_b,
           fc1_w, fc1_b, fc2_w, fc2_b):
    src = edge_index[0]
    dst = edge_index[1]
    px, pa = _sc_aggregate()(x, src, dst, edge_attr.reshape(-1), batch)

    hpad = 64
    w1xt = lin1_w[:, :D].T
    wa_ext = (jnp.zeros((D, INNER), jnp.float32)
              .at[:DE].set(lin1_w[:, D:].T)
              .at[DE].set(lin1_b))
    f1t = jnp.zeros((INNER, hpad), jnp.float32).at[:, :HID].set(fc1_w.T)
    b1f = jnp.zeros((1, hpad), jnp.float32).at[0, :HID].set(fc1_b)
    f2t = jnp.zeros((hpad, 128), jnp.float32).at[:HID, 0].set(fc2_w[0])
    b2r = jnp.zeros((1, 128), jnp.float32).at[0, 0].set(fc2_b[0])

    outf = pl.pallas_call(
        _tc_body,
        out_shape=jax.ShapeDtypeStruct((G, 128), jnp.float32),
    )(px, pa, w1xt, wa_ext, f1t, b1f, f2t, b2r)
    return outf[:, :1]


# speedup vs baseline: 12.9564x; 1.0001x over previous
import functools

import jax
import jax.numpy as jnp
from jax import lax
from jax.experimental import pallas as pl
from jax.experimental.pallas import tpu as pltpu
from jax.experimental.pallas import tpu_sc as plsc

N = 10000
E = 320000
D = 128
DE = 16
G = 512
INNER = 512
HID = 50

NC = 2
NS = 16
NW = NC * NS
EPW = E // NW
CH = 80
NCHUNK = EPW // CH
RPT = G // NS


def _sc_body(x_hbm, src_hbm, dst_hbm, eaf_hbm, batch_hbm,
             px_hbm, pa_hbm,
             ge_a, xrows_a, ealin_a,
             ge_b, xrows_b, ealin_b,
             src_all, dst_all,
             ea_v, zx_v, acc_x, acc_a, sem_a, sem_b):
    c = lax.axis_index("c")
    s = lax.axis_index("s")
    wid = s * NC + c

    zero16 = jnp.zeros((16,), jnp.float32)
    onehot16 = jnp.where(lax.iota(jnp.int32, 16) == 0, 1.0, 0.0)

    for j in range(RPT):
        for k in range(D // 16):
            zx_v[j, pl.ds(k * 16, 16)] = zero16
    for j in range(CH):
        ea_v[j, pl.ds(16, 16)] = onehot16
        for k in range(2, D // 16):
            ea_v[j, pl.ds(k * 16, 16)] = zero16

    rowbase = s * RPT
    pltpu.sync_copy(zx_v, acc_x.at[pl.ds(rowbase, RPT)])
    pltpu.sync_copy(zx_v, acc_a.at[pl.ds(rowbase, RPT)])
    ebase = wid * EPW
    pltpu.sync_copy(src_hbm.at[pl.ds(ebase, EPW)], src_all)
    pltpu.sync_copy(dst_hbm.at[pl.ds(ebase, EPW)], dst_all)
    plsc.subcore_barrier()

    bufs = ((ge_a, xrows_a, ealin_a, sem_a),
            (ge_b, xrows_b, ealin_b, sem_b))

    def load(j, buf):
        ge_v, xrows_v, ealin_v, sem = buf
        pltpu.async_copy(eaf_hbm.at[pl.ds((ebase + j * CH) * DE, CH * DE)],
                         ealin_v, sem)
        pltpu.async_copy(batch_hbm.at[dst_all.at[pl.ds(j * CH, CH)]], ge_v, sem)
        pltpu.async_copy(x_hbm.at[src_all.at[pl.ds(j * CH, CH)]], xrows_v, sem)

    def process(buf):
        ge_v, xrows_v, ealin_v, sem = buf
        pltpu.make_async_copy(eaf_hbm.at[pl.ds(0, CH * DE)], ealin_v, sem).wait()
        pltpu.make_async_copy(batch_hbm.at[ge_v], ge_v, sem).wait()
        pltpu.make_async_copy(x_hbm.at[ge_v], xrows_v, sem).wait()
        for e in range(CH):
            ea_v[e, pl.ds(0, 16)] = ealin_v[pl.ds(e * DE, 16)]
        pltpu.sync_copy(xrows_v, acc_x.at[ge_v], add=True)
        pltpu.sync_copy(ea_v, acc_a.at[ge_v], add=True)

    load(0, bufs[0])

    def pair(i2, carry):
        j0 = 2 * i2
        load(j0 + 1, bufs[1])
        process(bufs[0])

        @pl.when(j0 + 2 < NCHUNK)
        def _prefetch():
            load(j0 + 2, bufs[0])

        process(bufs[1])
        return carry

    lax.fori_loop(0, NCHUNK // 2, pair, 0)
    process(bufs[0])

    plsc.subcore_barrier()
    outbase = c * G + rowbase
    pltpu.sync_copy(acc_x.at[pl.ds(rowbase, RPT)], px_hbm.at[pl.ds(outbase, RPT)])
    pltpu.sync_copy(acc_a.at[pl.ds(rowbase, RPT)], pa_hbm.at[pl.ds(outbase, RPT)])


@functools.cache
def _sc_aggregate():
    return functools.partial(
        pl.kernel,
        out_type=[
            jax.ShapeDtypeStruct((NC * G, D), jnp.float32),
            jax.ShapeDtypeStruct((NC * G, D), jnp.float32),
        ],
        mesh=plsc.VectorSubcoreMesh(core_axis_name="c", subcore_axis_name="s"),
        compiler_params=pltpu.CompilerParams(needs_layout_passes=False),
        scratch_types=(
            [pltpu.VMEM((CH,), jnp.int32),
             pltpu.VMEM((CH, D), jnp.float32),
             pltpu.VMEM((CH * DE,), jnp.float32)]
            * 2
            + [pltpu.VMEM((EPW,), jnp.int32),
               pltpu.VMEM((EPW,), jnp.int32),
               pltpu.VMEM((CH, D), jnp.float32),
               pltpu.VMEM((RPT, D), jnp.float32),
               pltpu.VMEM_SHARED((G, D), jnp.float32),
               pltpu.VMEM_SHARED((G, D), jnp.float32),
               pltpu.SemaphoreType.DMA,
               pltpu.SemaphoreType.DMA]
        ),
    )(_sc_body)


def _tc_body(px, pa, w1, wa_ext, f1, f2t, b2r, out):
    def dotT(a, b):
        return lax.dot_general(a, b, (((1,), (1,)), ((), ())),
                               preferred_element_type=jnp.float32,
                               precision=lax.Precision.HIGHEST)

    dot = functools.partial(jnp.dot, preferred_element_type=jnp.float32,
                            precision=lax.Precision.HIGHEST)
    sx = px[0:G, :] + px[G:2 * G, :]
    sa = pa[0:G, :] + pa[G:2 * G, :]
    mol = dotT(sx, w1[:, 0:D]) + dotT(sa, wa_ext[...])
    hid = dotT(mol, f1[...])
    out[...] = dot(hid, f2t[...]) + b2r[...]


def kernel(x, edge_index, edge_attr, smiles, batch, lin1_w, lin1_b,
           fc1_w, fc1_b, fc2_w, fc2_b):
    src = edge_index[0]
    dst = edge_index[1]
    px, pa = _sc_aggregate()(x, src, dst, edge_attr.reshape(-1), batch)

    wa_ext = jnp.concatenate(
        [lin1_w[:, D:], lin1_b[:, None],
         jnp.zeros((INNER, D - DE - 1), jnp.float32)], axis=1)
    f2t = jnp.zeros((HID, 128), jnp.float32).at[:, 0].set(fc2_w[0])
    b2r = (jnp.zeros((1, 128), jnp.float32)
           .at[0, 0].set(fc2_b[0] + fc2_w[0] @ fc1_b))

    outf = pl.pallas_call(
        _tc_body,
        out_shape=jax.ShapeDtypeStruct((G, 128), jnp.float32),
    )(px, pa, lin1_w, wa_ext, fc1_w, f2t, b2r)
    return outf[:, :1]
